# baseline (speedup 1.0000x reference)
import jax
import jax.numpy as jnp
from jax.experimental import pallas as pl
from jax.experimental.pallas import tpu as pltpu
from jax.experimental.pallas import tpu_sc as plsc

N = 2048
HID = 1024
CODE = 256
CONF = 64
E = 8
BLK = 256
NBLK = N // BLK
BS = 256
NBMAX = 2 * N // BS + E
SROWS = NBMAX * BS
W = 64
HPK = HID // 2

_f32 = jnp.float32
_bf16 = jnp.bfloat16


def _dot(a, b):
    return jax.lax.dot_general(a, b, (((1,), (0,)), ((), ())),
                               preferred_element_type=_f32)


def _dot_t(a, b):
    return jax.lax.dot_general(a, b, (((1,), (1,)), ((), ())),
                               preferred_element_type=_f32)


def _pack(x):
    lo = jax.lax.bitcast_convert_type(x[:, :HPK].astype(_bf16), jnp.uint16)
    hi = jax.lax.bitcast_convert_type(x[:, HPK:].astype(_bf16), jnp.uint16)
    packed = lo.astype(jnp.uint32) | (hi.astype(jnp.uint32) << 16)
    return jax.lax.bitcast_convert_type(packed, jnp.int32)


def _unpack(p):
    u = jax.lax.bitcast_convert_type(p, jnp.uint32)
    lo = jax.lax.bitcast_convert_type(u << 16, _f32)
    hi = jax.lax.bitcast_convert_type(u & jnp.uint32(0xFFFF0000), _f32)
    return jnp.concatenate([lo, hi], axis=1)


def _router_body(code_ref, rtW_ref, rt_b_ref,
                 pT_ref, g1_ref, g2_ref, be_ref, used_ref, aux_ref):
    logits = _dot_t(code_ref[...], rtW_ref[...]) + rt_b_ref[...]
    mx = jnp.max(logits, axis=1, keepdims=True)
    exl = jnp.exp(logits - mx)
    probs = exl / jnp.sum(exl, axis=1, keepdims=True)
    lane = jax.lax.broadcasted_iota(jnp.int32, (N, E), 1)
    m1 = jnp.max(probs, axis=1, keepdims=True)
    e1 = jnp.min(jnp.where(probs >= m1, lane, E), axis=1, keepdims=True)
    oh1 = lane == e1
    pn1 = jnp.where(oh1, -1.0, probs)
    m2 = jnp.max(pn1, axis=1, keepdims=True)
    e2 = jnp.min(jnp.where(pn1 >= m2, lane, E), axis=1, keepdims=True)
    oh2 = lane == e2
    maskD = (probs >= m2).astype(_f32)

    den = m1 + m2 + 1e-9
    g1_ref[...] = m1 / den
    g2_ref[...] = m2 / den

    cnt = oh1.astype(_f32) + oh2.astype(_f32)
    inc = cnt
    k = 1
    while k < N:
        inc = inc + jnp.concatenate(
            [jnp.zeros((k, E), _f32), inc[:-k, :]], axis=0)
        k *= 2
    ecs = inc - cnt
    totals = inc[N - 1:N, :]
    pblk = jnp.ceil(totals * (1.0 / BS))
    incb = pblk
    k = 1
    while k < E:
        incb = incb + jnp.concatenate(
            [jnp.zeros((1, k), _f32), incb[:, :-k]], axis=1)
        k *= 2
    base_rows = (incb - pblk) * BS

    rank1 = jnp.sum(jnp.where(oh1, ecs, 0.0), axis=1, keepdims=True)
    rank2 = jnp.sum(jnp.where(oh2, ecs, 0.0), axis=1, keepdims=True)
    b1 = jnp.sum(jnp.where(oh1, base_rows, 0.0), axis=1, keepdims=True)
    b2 = jnp.sum(jnp.where(oh2, base_rows, 0.0), axis=1, keepdims=True)
    p12 = jnp.concatenate([b1 + rank1, b2 + rank2], axis=1)
    pT_ref[...] = jnp.transpose(p12, (1, 0)).astype(jnp.int32)

    bi = jax.lax.broadcasted_iota(jnp.int32, (1, NBMAX), 1).astype(_f32)
    be = jnp.zeros((1, NBMAX), _f32)
    for e in range(E):
        be += (bi >= incb[0:1, e:e + 1]).astype(_f32)
    be_ref[...] = jnp.minimum(be, E - 1).astype(jnp.int32)
    used_ref[...] = incb[0:1, E - 1:E].astype(jnp.int32)

    pm = jnp.sum(probs, axis=0, keepdims=True)
    mm = jnp.sum(maskD, axis=0, keepdims=True)
    aux_ref[...] = (E / (N * N)) * jnp.sum(pm * mm, axis=(0, 1), keepdims=True)


def _dense_body(code_ref, u_ref, h_ref,
                scW1c_ref, scW1u_ref, scW2_ref,
                shW1c_ref, shW1u_ref, shW2_ref, pj_ref,
                sc_b1_ref, sc_b2_ref, sh_b1_ref, sh_b2_ref, pj_b_ref,
                hm_ref):
    code = code_ref[...]
    u = u_ref[...]
    t = _dot(code, scW1c_ref[...]) + _dot(u, scW1u_ref[...]) + sc_b1_ref[...]
    t = t * jax.nn.sigmoid(t)
    scale = jax.nn.sigmoid(_dot(t.astype(_bf16), scW2_ref[...]) + sc_b2_ref[...])
    s = _dot(code, shW1c_ref[...]) + _dot(u, shW1u_ref[...]) + sh_b1_ref[...]
    s = s * jax.nn.sigmoid(s)
    shift = _dot(s.astype(_bf16), shW2_ref[...]) + sh_b2_ref[...]
    ht = jnp.maximum(_dot(h_ref[...], pj_ref[...]) + pj_b_ref[...], 0.0)
    hm_ref[...] = _pack(scale * ht + shift)


def _conf_body(u_ref, fcW1_ref, fcW2_ref, fc_b1_ref, fc_b2_ref, cmask_ref,
               ce_ref):
    ce = _dot(jnp.maximum(_dot(u_ref[...], fcW1_ref[...]) + fc_b1_ref[...],
                          0.0).astype(_bf16), fcW2_ref[...]) + fc_b2_ref[...]
    ce_ref[...] = ce * (cmask_ref[...] > 0.0).astype(_f32)


def _expert_body(be_ref, used_ref, x_ref, w_ref, b_ref, o_ref,
                 wb_ref, laste_ref):
    i = pl.program_id(0)
    e = be_ref[0, i]

    @pl.when((i == 0) | (e != laste_ref[0]))
    def _():
        wb_ref[...] = w_ref[0].astype(_bf16)
        laste_ref[0] = e

    @pl.when(i < used_ref[0, 0])
    def _():
        xb = _unpack(x_ref[...]).astype(_bf16)
        o_ref[...] = _pack(jnp.maximum(
            jax.lax.dot_general(xb, wb_ref[...],
                                (((1,), (1,)), ((), ())),
                                preferred_element_type=_f32) + b_ref[0], 0.0))


def _combine_body(ga_ref, gb_ref, g1_ref, g2_ref, ce_ref, cmask_ref,
                  out_ref):
    moe = (g1_ref[...] * _unpack(ga_ref[...])
           + g2_ref[...] * _unpack(gb_ref[...]))
    out_ref[...] = (moe * (1.0 - jax.nn.sigmoid(cmask_ref[...]))
                    + ce_ref[...])


def _sc_scatter(hm, pall):
    mesh = plsc.VectorSubcoreMesh(core_axis_name="c", subcore_axis_name="s")

    @pl.kernel(out_type=jax.ShapeDtypeStruct((SROWS, HPK), jnp.int32), mesh=mesh)
    def k(hm_hbm, i_hbm, o_hbm):
        def body(x_vmem, i_vmem):
            pltpu.sync_copy(x_vmem, o_hbm.at[i_vmem])

        pltpu.emit_pipeline(
            body,
            grid=(2 * N // W,),
            in_specs=[pl.BlockSpec((W, HPK), lambda i: (i % (N // W), 0)),
                      pl.BlockSpec((W,), lambda i: (i,))],
            out_specs=[],
            core_axis_name=("c", "s"),
            dimension_semantics=(pltpu.PARALLEL,),
        )(hm_hbm, i_hbm)

    return k(hm, pall)


def _sc_gather(res, pall):
    mesh = plsc.VectorSubcoreMesh(core_axis_name="c", subcore_axis_name="s")

    @pl.kernel(out_type=jax.ShapeDtypeStruct((2 * N, HPK), jnp.int32), mesh=mesh)
    def k(res_hbm, i_hbm, o_hbm):
        def body(i_vmem, o_vmem):
            pltpu.sync_copy(res_hbm.at[i_vmem], o_vmem)

        pltpu.emit_pipeline(
            body,
            grid=(2 * N // W,),
            in_specs=[pl.BlockSpec((W,), lambda i: (i,))],
            out_specs=[pl.BlockSpec((W, HPK), lambda i: (i, 0))],
            core_axis_name=("c", "s"),
            dimension_semantics=(pltpu.PARALLEL,),
        )(i_hbm, o_hbm)

    return k(res, pall)


def kernel(h, code_emb, u, conf_mask, fc_W1, fc_b1, fc_W2, fc_b2,
           sc_W1, sc_b1, sc_W2, sc_b2, sh_W1, sh_b1, sh_W2, sh_b2,
           pj_W, pj_b, rt_W, rt_b, ex_W, ex_b):
    bf = _bf16
    row = lambda v: v.reshape(1, -1).astype(_f32)
    codeb = code_emb.astype(bf)
    ub = u.astype(bf)
    hb = h.astype(bf)

    blk = lambda shape, im: pl.BlockSpec(shape, im)
    tok = lambda d: blk((BLK, d), lambda i: (i, 0))
    cst = lambda shape: blk(shape, lambda i: tuple(0 for _ in shape))

    pT, g1, g2, be, used, aux = pl.pallas_call(
        _router_body,
        out_shape=[jax.ShapeDtypeStruct((2, N), jnp.int32),
                   jax.ShapeDtypeStruct((N, 1), _f32),
                   jax.ShapeDtypeStruct((N, 1), _f32),
                   jax.ShapeDtypeStruct((1, NBMAX), jnp.int32),
                   jax.ShapeDtypeStruct((1, 1), jnp.int32),
                   jax.ShapeDtypeStruct((1, 1), _f32)],
    )(codeb, rt_W.astype(bf), row(rt_b))

    hm = pl.pallas_call(
        _dense_body,
        grid=(NBLK,),
        in_specs=[tok(CODE), tok(CONF), tok(HID),
                  cst((CODE, HID)), cst((CONF, HID)), cst((HID, HID)),
                  cst((CODE, HID)), cst((CONF, HID)), cst((HID, HID)),
                  cst((HID, HID)),
                  cst((1, HID)), cst((1, HID)), cst((1, HID)), cst((1, HID)),
                  cst((1, HID))],
        out_specs=tok(HPK),
        out_shape=jax.ShapeDtypeStruct((N, HPK), jnp.int32),
    )(codeb, ub, hb,
      sc_W1[:, :CODE].T.astype(bf), sc_W1[:, CODE:].T.astype(bf),
      sc_W2.T.astype(bf),
      sh_W1[:, :CODE].T.astype(bf), sh_W1[:, CODE:].T.astype(bf),
      sh_W2.T.astype(bf), pj_W.T.astype(bf),
      row(sc_b1), row(sc_b2), row(sh_b1), row(sh_b2), row(pj_b))

    pall = pT.reshape(2 * N)
    hs = _sc_scatter(hm, pall)

    ce = pl.pallas_call(
        _conf_body,
        grid=(NBLK,),
        in_specs=[tok(CONF), cst((CONF, HID)), cst((HID, HID)),
                  cst((1, HID)), cst((1, HID)), cst((1, HID))],
        out_specs=tok(HID),
        out_shape=jax.ShapeDtypeStruct((N, HID), _f32),
    )(ub, fc_W1.T.astype(bf), fc_W2.T.astype(bf),
      row(fc_b1), row(fc_b2), row(conf_mask))

    res = pl.pallas_call(
        _expert_body,
        grid_spec=pltpu.PrefetchScalarGridSpec(
            num_scalar_prefetch=2,
            grid=(NBMAX,),
            in_specs=[
                pl.BlockSpec((BS, HPK),
                             lambda i, be, us: (jnp.minimum(i, us[0, 0] - 1), 0)),
                pl.BlockSpec((1, HID, HID),
                             lambda i, be, us: (be[0, jnp.minimum(i, us[0, 0] - 1)], 0, 0)),
                pl.BlockSpec((1, 1, HID),
                             lambda i, be, us: (be[0, jnp.minimum(i, us[0, 0] - 1)], 0, 0)),
            ],
            out_specs=pl.BlockSpec(
                (BS, HPK), lambda i, be, us: (jnp.minimum(i, us[0, 0] - 1), 0)),
            scratch_shapes=[pltpu.VMEM((HID, HID), _bf16),
                            pltpu.SMEM((1,), jnp.int32)],
        ),
        out_shape=jax.ShapeDtypeStruct((SROWS, HPK), jnp.int32),
    )(be, used, hs, ex_W, ex_b.reshape(E, 1, HID).astype(_f32))

    g = _sc_gather(res, pall)

    out = pl.pallas_call(
        _combine_body,
        grid=(NBLK,),
        in_specs=[blk((BLK, HPK), lambda i: (i, 0)),
                  blk((BLK, HPK), lambda i: (i + NBLK, 0)),
                  blk((BLK, 1), lambda i: (i, 0)),
                  blk((BLK, 1), lambda i: (i, 0)),
                  tok(HID), cst((1, HID))],
        out_specs=tok(HID),
        out_shape=jax.ShapeDtypeStruct((N, HID), _f32),
    )(g, g, g1, g2, ce, row(conf_mask))

    return out, aux.reshape(())


# speedup vs baseline: 1.0314x; 1.0314x over previous
import jax
import jax.numpy as jnp
from jax.experimental import pallas as pl
from jax.experimental.pallas import tpu as pltpu
from jax.experimental.pallas import tpu_sc as plsc

N = 2048
HID = 1024
CODE = 256
CONF = 64
E = 8
BLK = 256
NBLK = N // BLK
BS = 256
NBMAX = 2 * N // BS + E
SROWS = NBMAX * BS
W = 64
HPK = HID // 2

_f32 = jnp.float32
_bf16 = jnp.bfloat16


def _dot(a, b):
    return jax.lax.dot_general(a, b, (((1,), (0,)), ((), ())),
                               preferred_element_type=_f32)


def _dot_t(a, b):
    return jax.lax.dot_general(a, b, (((1,), (1,)), ((), ())),
                               preferred_element_type=_f32)


def _pack(x):
    lo = jax.lax.bitcast_convert_type(x[:, :HPK].astype(_bf16), jnp.uint16)
    hi = jax.lax.bitcast_convert_type(x[:, HPK:].astype(_bf16), jnp.uint16)
    packed = lo.astype(jnp.uint32) | (hi.astype(jnp.uint32) << 16)
    return jax.lax.bitcast_convert_type(packed, jnp.int32)


def _unpack(p):
    u = jax.lax.bitcast_convert_type(p, jnp.uint32)
    lo = jax.lax.bitcast_convert_type(u << 16, _f32)
    hi = jax.lax.bitcast_convert_type(u & jnp.uint32(0xFFFF0000), _f32)
    return jnp.concatenate([lo, hi], axis=1)


def _router_body(code_ref, rtW_ref, rt_b_ref,
                 pT_ref, g1_ref, g2_ref, be_ref, used_ref, aux_ref):
    logits = _dot_t(code_ref[...], rtW_ref[...]) + rt_b_ref[...]
    mx = jnp.max(logits, axis=1, keepdims=True)
    exl = jnp.exp(logits - mx)
    probs = exl / jnp.sum(exl, axis=1, keepdims=True)
    lane = jax.lax.broadcasted_iota(jnp.int32, (N, E), 1)
    m1 = jnp.max(probs, axis=1, keepdims=True)
    e1 = jnp.min(jnp.where(probs >= m1, lane, E), axis=1, keepdims=True)
    oh1 = lane == e1
    pn1 = jnp.where(oh1, -1.0, probs)
    m2 = jnp.max(pn1, axis=1, keepdims=True)
    e2 = jnp.min(jnp.where(pn1 >= m2, lane, E), axis=1, keepdims=True)
    oh2 = lane == e2
    maskD = (probs >= m2).astype(_f32)

    den = m1 + m2 + 1e-9
    g1_ref[...] = m1 / den
    g2_ref[...] = m2 / den

    cnt = oh1.astype(_f32) + oh2.astype(_f32)
    inc = cnt
    k = 1
    while k < N:
        inc = inc + jnp.concatenate(
            [jnp.zeros((k, E), _f32), inc[:-k, :]], axis=0)
        k *= 2
    ecs = inc - cnt
    totals = inc[N - 1:N, :]
    pblk = jnp.ceil(totals * (1.0 / BS))
    incb = pblk
    k = 1
    while k < E:
        incb = incb + jnp.concatenate(
            [jnp.zeros((1, k), _f32), incb[:, :-k]], axis=1)
        k *= 2
    base_rows = (incb - pblk) * BS

    rank1 = jnp.sum(jnp.where(oh1, ecs, 0.0), axis=1, keepdims=True)
    rank2 = jnp.sum(jnp.where(oh2, ecs, 0.0), axis=1, keepdims=True)
    b1 = jnp.sum(jnp.where(oh1, base_rows, 0.0), axis=1, keepdims=True)
    b2 = jnp.sum(jnp.where(oh2, base_rows, 0.0), axis=1, keepdims=True)
    p12 = jnp.concatenate([b1 + rank1, b2 + rank2], axis=1)
    pT_ref[...] = jnp.transpose(p12, (1, 0)).astype(jnp.int32)

    bi = jax.lax.broadcasted_iota(jnp.int32, (1, NBMAX), 1).astype(_f32)
    be = jnp.zeros((1, NBMAX), _f32)
    for e in range(E):
        be += (bi >= incb[0:1, e:e + 1]).astype(_f32)
    be_ref[...] = jnp.minimum(be, E - 1).astype(jnp.int32)
    used_ref[...] = incb[0:1, E - 1:E].astype(jnp.int32)

    pm = jnp.sum(probs, axis=0, keepdims=True)
    mm = jnp.sum(maskD, axis=0, keepdims=True)
    aux_ref[...] = (E / (N * N)) * jnp.sum(pm * mm, axis=(0, 1), keepdims=True)


def _dense_body(code_ref, u_ref, h_ref,
                scW1_ref, scW2_ref, shW1_ref, shW2_ref, pj_ref,
                sc_b1_ref, sc_b2_ref, sh_b1_ref, sh_b2_ref, pj_b_ref,
                hm_ref):
    code = code_ref[...]
    u = u_ref[...]
    t = (_dot_t(code, scW1_ref[:, :CODE]) + _dot_t(u, scW1_ref[:, CODE:])
         + sc_b1_ref[...])
    t = t * jax.nn.sigmoid(t)
    scale = jax.nn.sigmoid(_dot_t(t.astype(_bf16), scW2_ref[...]) + sc_b2_ref[...])
    s = (_dot_t(code, shW1_ref[:, :CODE]) + _dot_t(u, shW1_ref[:, CODE:])
         + sh_b1_ref[...])
    s = s * jax.nn.sigmoid(s)
    shift = _dot_t(s.astype(_bf16), shW2_ref[...]) + sh_b2_ref[...]
    ht = jnp.maximum(_dot_t(h_ref[...], pj_ref[...]) + pj_b_ref[...], 0.0)
    hm_ref[...] = _pack(scale * ht + shift)


def _conf_body(u_ref, fcW1_ref, fcW2_ref, fc_b1_ref, fc_b2_ref, cmask_ref,
               ce_ref):
    ce = _dot_t(jnp.maximum(_dot_t(u_ref[...], fcW1_ref[...]) + fc_b1_ref[...],
                            0.0).astype(_bf16), fcW2_ref[...]) + fc_b2_ref[...]
    ce_ref[...] = ce * (cmask_ref[...] > 0.0).astype(_f32)


def _expert_body(be_ref, used_ref, x_ref, w_ref, b_ref, o_ref):
    i = pl.program_id(0)

    @pl.when(i < used_ref[0, 0])
    def _():
        xb = _unpack(x_ref[...]).astype(_bf16)
        o_ref[...] = _pack(jnp.maximum(
            jax.lax.dot_general(xb, w_ref[0].astype(_bf16),
                                (((1,), (1,)), ((), ())),
                                preferred_element_type=_f32) + b_ref[0], 0.0))


def _combine_body(ga_ref, gb_ref, g1_ref, g2_ref, ce_ref, cmask_ref,
                  out_ref):
    moe = (g1_ref[...] * _unpack(ga_ref[...])
           + g2_ref[...] * _unpack(gb_ref[...]))
    out_ref[...] = (moe * (1.0 - jax.nn.sigmoid(cmask_ref[...]))
                    + ce_ref[...])


def _sc_scatter(hm, pall):
    mesh = plsc.VectorSubcoreMesh(core_axis_name="c", subcore_axis_name="s")

    @pl.kernel(out_type=jax.ShapeDtypeStruct((SROWS, HPK), jnp.int32), mesh=mesh)
    def k(hm_hbm, i_hbm, o_hbm):
        def body(x_vmem, i_vmem):
            pltpu.sync_copy(x_vmem, o_hbm.at[i_vmem])

        pltpu.emit_pipeline(
            body,
            grid=(2 * N // W,),
            in_specs=[pl.BlockSpec((W, HPK), lambda i: (i % (N // W), 0)),
                      pl.BlockSpec((W,), lambda i: (i,))],
            out_specs=[],
            core_axis_name=("c", "s"),
            dimension_semantics=(pltpu.PARALLEL,),
        )(hm_hbm, i_hbm)

    return k(hm, pall)


def _sc_gather(res, pall):
    mesh = plsc.VectorSubcoreMesh(core_axis_name="c", subcore_axis_name="s")

    @pl.kernel(out_type=jax.ShapeDtypeStruct((2 * N, HPK), jnp.int32), mesh=mesh)
    def k(res_hbm, i_hbm, o_hbm):
        def body(i_vmem, o_vmem):
            pltpu.sync_copy(res_hbm.at[i_vmem], o_vmem)

        pltpu.emit_pipeline(
            body,
            grid=(2 * N // W,),
            in_specs=[pl.BlockSpec((W,), lambda i: (i,))],
            out_specs=[pl.BlockSpec((W, HPK), lambda i: (i, 0))],
            core_axis_name=("c", "s"),
            dimension_semantics=(pltpu.PARALLEL,),
        )(i_hbm, o_hbm)

    return k(res, pall)


def kernel(h, code_emb, u, conf_mask, fc_W1, fc_b1, fc_W2, fc_b2,
           sc_W1, sc_b1, sc_W2, sc_b2, sh_W1, sh_b1, sh_W2, sh_b2,
           pj_W, pj_b, rt_W, rt_b, ex_W, ex_b):
    bf = _bf16
    row = lambda v: v.reshape(1, -1).astype(_f32)
    codeb = code_emb.astype(bf)
    ub = u.astype(bf)
    hb = h.astype(bf)

    blk = lambda shape, im: pl.BlockSpec(shape, im)
    tok = lambda d: blk((BLK, d), lambda i: (i, 0))
    cst = lambda shape: blk(shape, lambda i: tuple(0 for _ in shape))

    pT, g1, g2, be, used, aux = pl.pallas_call(
        _router_body,
        out_shape=[jax.ShapeDtypeStruct((2, N), jnp.int32),
                   jax.ShapeDtypeStruct((N, 1), _f32),
                   jax.ShapeDtypeStruct((N, 1), _f32),
                   jax.ShapeDtypeStruct((1, NBMAX), jnp.int32),
                   jax.ShapeDtypeStruct((1, 1), jnp.int32),
                   jax.ShapeDtypeStruct((1, 1), _f32)],
    )(codeb, rt_W.astype(bf), row(rt_b))

    hm = pl.pallas_call(
        _dense_body,
        grid=(NBLK,),
        in_specs=[tok(CODE), tok(CONF), tok(HID),
                  cst((HID, CODE + CONF)), cst((HID, HID)),
                  cst((HID, CODE + CONF)), cst((HID, HID)),
                  cst((HID, HID)),
                  cst((1, HID)), cst((1, HID)), cst((1, HID)), cst((1, HID)),
                  cst((1, HID))],
        out_specs=tok(HPK),
        out_shape=jax.ShapeDtypeStruct((N, HPK), jnp.int32),
    )(codeb, ub, hb,
      sc_W1.astype(bf), sc_W2.astype(bf),
      sh_W1.astype(bf), sh_W2.astype(bf), pj_W.astype(bf),
      row(sc_b1), row(sc_b2), row(sh_b1), row(sh_b2), row(pj_b))

    pall = pT.reshape(2 * N)
    hs = _sc_scatter(hm, pall)

    ce = pl.pallas_call(
        _conf_body,
        grid=(NBLK,),
        in_specs=[tok(CONF), cst((HID, CONF)), cst((HID, HID)),
                  cst((1, HID)), cst((1, HID)), cst((1, HID))],
        out_specs=tok(HID),
        out_shape=jax.ShapeDtypeStruct((N, HID), _f32),
    )(ub, fc_W1.astype(bf), fc_W2.astype(bf),
      row(fc_b1), row(fc_b2), row(conf_mask))

    res = pl.pallas_call(
        _expert_body,
        grid_spec=pltpu.PrefetchScalarGridSpec(
            num_scalar_prefetch=2,
            grid=(NBMAX,),
            in_specs=[
                pl.BlockSpec((BS, HPK),
                             lambda i, be, us: (jnp.minimum(i, us[0, 0] - 1), 0)),
                pl.BlockSpec((1, HID, HID),
                             lambda i, be, us: (be[0, jnp.minimum(i, us[0, 0] - 1)], 0, 0)),
                pl.BlockSpec((1, 1, HID),
                             lambda i, be, us: (be[0, jnp.minimum(i, us[0, 0] - 1)], 0, 0)),
            ],
            out_specs=pl.BlockSpec(
                (BS, HPK), lambda i, be, us: (jnp.minimum(i, us[0, 0] - 1), 0)),
        ),
        out_shape=jax.ShapeDtypeStruct((SROWS, HPK), jnp.int32),
    )(be, used, hs, ex_W, ex_b.reshape(E, 1, HID).astype(_f32))

    g = _sc_gather(res, pall)

    out = pl.pallas_call(
        _combine_body,
        grid=(NBLK,),
        in_specs=[blk((BLK, HPK), lambda i: (i, 0)),
                  blk((BLK, HPK), lambda i: (i + NBLK, 0)),
                  blk((BLK, 1), lambda i: (i, 0)),
                  blk((BLK, 1), lambda i: (i, 0)),
                  tok(HID), cst((1, HID))],
        out_specs=tok(HID),
        out_shape=jax.ShapeDtypeStruct((N, HID), _f32),
    )(g, g, g1, g2, ce, row(conf_mask))

    return out, aux.reshape(())
